# baseline (speedup 1.0000x reference)
import functools

import jax
import jax.numpy as jnp
import numpy as np
from jax import lax
from jax.experimental import pallas as pl
from jax.experimental.pallas import tpu as pltpu
from jax.experimental.pallas import tpu_sc as plsc

_NVARS = 99
_SHAPE = (8, 40320, _NVARS)
_NW = 32
_WPP = 4
_ROWS_W = _SHAPE[1] // _WPP
_RCHUNK = 240
_NCHUNK = _ROWS_W // _RCHUNK
_NTRIP = _NCHUNK // 3
_TAIL = 83


def _affine_consts():
    scale = np.ones(_NVARS, dtype=np.float64)
    shift = np.zeros(_NVARS, dtype=np.float64)
    for i, m in zip((10, 11, 12), (11.0, 12.0, 13.0)):
        scale[i] = 1.0 / m
    i = np.arange(13, _NVARS).astype(np.float64)
    mu = (0.1 * i).astype(np.float32).astype(np.float64)
    sd = (1.0 + 0.01 * i).astype(np.float32).astype(np.float64)
    scale[13:] = 1.0 / sd
    shift[13:] = -(mu / sd)
    return jnp.asarray(scale.astype(np.float32)), jnp.asarray(shift.astype(np.float32))


def _sc_norm(x, scp, shp):
    mesh = plsc.VectorSubcoreMesh(core_axis_name="c", subcore_axis_name="s")

    @functools.partial(
        pl.kernel,
        mesh=mesh,
        out_type=jax.ShapeDtypeStruct(_SHAPE, jnp.float32),
        scratch_types=[
            pltpu.VMEM((_RCHUNK, _NVARS), jnp.float32),
            pltpu.VMEM((_RCHUNK, _NVARS), jnp.float32),
            pltpu.VMEM((_RCHUNK, _NVARS), jnp.float32),
            pltpu.VMEM((_NVARS,), jnp.float32),
            pltpu.VMEM((_NVARS,), jnp.float32),
            pltpu.SemaphoreType.DMA,
            pltpu.SemaphoreType.DMA,
            pltpu.SemaphoreType.DMA,
            pltpu.SemaphoreType.DMA,
            pltpu.SemaphoreType.DMA,
            pltpu.SemaphoreType.DMA,
        ],
        compiler_params=pltpu.CompilerParams(
            use_tc_tiling_on_sc=True, needs_layout_passes=True),
    )
    def k(x_hbm, scp_hbm, shp_hbm, out_hbm,
          b0, b1, b2, scv, shv, si0, si1, si2, so0, so1, so2):
        wid = lax.axis_index("s") * 2 + lax.axis_index("c")
        d0 = wid // _WPP
        r0 = (wid % _WPP) * _ROWS_W
        bufs = (b0, b1, b2)
        sis = (si0, si1, si2)
        sos = (so0, so1, so2)

        pltpu.sync_copy(scp_hbm, scv)
        pltpu.sync_copy(shp_hbm, shv)

        starts = [16 * j for j in range(6)] + [_TAIL]
        avs = [scv[pl.ds(s, 16)] for s in starts]
        bvs = [shv[pl.ds(s, 16)] for s in starts]

        def in_slice(c):
            return x_hbm.at[d0, pl.ds(r0 + c * _RCHUNK, _RCHUNK), :]

        def out_slice(c):
            return out_hbm.at[d0, pl.ds(r0 + c * _RCHUNK, _RCHUNK), :]

        def compute(buf):
            @plsc.parallel_loop(0, _RCHUNK, step=1, unroll=8)
            def _(row):
                vals = [buf[row, pl.ds(s, 16)] for s in starts]
                for (s, v, a, b) in zip(starts, vals, avs, bvs):
                    buf[row, pl.ds(s, 16)] = v * a + b

        pltpu.async_copy(in_slice(0), b0, si0)
        pltpu.async_copy(in_slice(1), b1, si1)

        def triple(t, carry):
            for b in range(3):
                c = t * 3 + b
                buf, si, so = bufs[b], sis[b], sos[b]
                pb = (b - 1) % 3
                pbuf, psi, pso = bufs[pb], sis[pb], sos[pb]

                pltpu.make_async_copy(in_slice(c), buf, si).wait()

                if b == 0:
                    @pl.when(t >= 1)
                    def _():
                        pltpu.make_async_copy(pbuf, out_slice(c - 1), pso).wait()

                    pltpu.async_copy(in_slice(c + 2), pbuf, psi)
                else:
                    pltpu.make_async_copy(pbuf, out_slice(c - 1), pso).wait()

                    @pl.when(t < _NTRIP - 1)
                    def _():
                        pltpu.async_copy(in_slice(c + 2), pbuf, psi)

                compute(buf)
                pltpu.async_copy(buf, out_slice(c), so)
            return carry

        lax.fori_loop(0, _NTRIP, triple, 0)
        pltpu.make_async_copy(b2, out_slice(_NCHUNK - 1), so2).wait()

    return k(x, scp, shp)


@functools.partial(jax.jit)
def kernel(x):
    scp, shp = _affine_consts()
    return _sc_norm(x, scp, shp)


# speedup vs baseline: 3.6758x; 3.6758x over previous
import functools

import jax
import jax.numpy as jnp
from jax import lax
from jax.experimental import pallas as pl
from jax.experimental.pallas import tpu as pltpu
from jax.experimental.pallas import tpu_sc as plsc

_NVARS = 99
_SHAPE = (8, 40320, _NVARS)
_R = _SHAPE[0]
_M = _SHAPE[1]
_NW = 32
_CHUNK_L = 21 * 128
_MCH = _M // _CHUNK_L
_ITEMS = _NVARS * _MCH
_NBUF = 4
_NGRP = 12
_NVEC = _CHUNK_L // 16


def _sc_norm(xt):
    mesh = plsc.VectorSubcoreMesh(core_axis_name="c", subcore_axis_name="s")

    @functools.partial(
        pl.kernel,
        mesh=mesh,
        out_type=jax.ShapeDtypeStruct((_NVARS, _R, _M), jnp.float32),
        scratch_types=[
            pltpu.VMEM((_R, _CHUNK_L), jnp.float32),
            pltpu.VMEM((_R, _CHUNK_L), jnp.float32),
            pltpu.VMEM((_R, _CHUNK_L), jnp.float32),
            pltpu.VMEM((_R, _CHUNK_L), jnp.float32),
            pltpu.SemaphoreType.DMA,
            pltpu.SemaphoreType.DMA,
            pltpu.SemaphoreType.DMA,
            pltpu.SemaphoreType.DMA,
            pltpu.SemaphoreType.DMA,
            pltpu.SemaphoreType.DMA,
            pltpu.SemaphoreType.DMA,
            pltpu.SemaphoreType.DMA,
        ],
        compiler_params=pltpu.CompilerParams(use_tc_tiling_on_sc=True),
    )
    def k(x_hbm, out_hbm, b0, b1, b2, b3,
          si0, si1, si2, si3, so0, so1, so2, so3):
        w = lax.axis_index("s") * 2 + lax.axis_index("c")
        bufs = (b0, b1, b2, b3)
        sis = (si0, si1, si2, si3)
        sos = (so0, so1, so2, so3)

        def item(s):
            return s * _NW + w

        def valid(s):
            return item(s) < _ITEMS

        def in_sl(s):
            i = item(s)
            return x_hbm.at[i // _MCH, :, pl.ds((i % _MCH) * _CHUNK_L, _CHUNK_L)]

        def out_sl(s):
            i = item(s)
            return out_hbm.at[i // _MCH, :, pl.ds((i % _MCH) * _CHUNK_L, _CHUNK_L)]

        def coeffs(s):
            ci = item(s) // _MCH
            cf = ci.astype(jnp.float32)
            denom = jnp.where(ci < 10, 1.0,
                              jnp.where(ci < 13, cf + 1.0, 0.01 * cf + 1.0))
            mufac = jnp.where(ci < 13, 0.0, 0.1 * cf)
            dv = lax.broadcast(denom, (16,))
            av = jnp.ones((16,), jnp.float32) / dv
            bv = -lax.broadcast(mufac, (16,)) * av
            return av, bv

        def compute(buf, s):
            av, bv = coeffs(s)

            @plsc.parallel_loop(0, _R, step=1, unroll=2)
            def _(r):
                for j in range(_NVEC):
                    sl = pl.ds(j * 16, 16)
                    buf[r, sl] = buf[r, sl] * av + bv

        for p in range(_NBUF - 1):
            pltpu.async_copy(in_sl(p), bufs[p], sis[p])

        def group(t, carry):
            for b in range(_NBUF):
                s = t * _NBUF + b
                buf, si, so = bufs[b], sis[b], sos[b]
                pb = (b - 1) % _NBUF
                pbuf, psi, pso = bufs[pb], sis[pb], sos[pb]

                @pl.when(valid(s))
                def _():
                    pltpu.make_async_copy(in_sl(s), buf, si).wait()

                if b == 0:
                    prev_ok = jnp.logical_and(t >= 1, valid(s - 1))
                else:
                    prev_ok = valid(s - 1)

                @pl.when(prev_ok)
                def _():
                    pltpu.make_async_copy(pbuf, out_sl(s - 1), pso).wait()

                @pl.when(valid(s + _NBUF - 1))
                def _():
                    pltpu.async_copy(in_sl(s + _NBUF - 1), pbuf, psi)

                @pl.when(valid(s))
                def _():
                    compute(buf, s)
                    pltpu.async_copy(buf, out_sl(s), so)
            return carry

        lax.fori_loop(0, _NGRP, group, 0)

    return k(xt)


@functools.partial(jax.jit)
def kernel(x):
    xt = jnp.transpose(x, (2, 0, 1))
    out_t = _sc_norm(xt)
    return jnp.transpose(out_t, (1, 2, 0))
